# Initial kernel scaffold; baseline (speedup 1.0000x reference)
import jax
import jax.numpy as jnp
from jax.experimental import pallas as pl


def kernel(node_features, edge_features, sub_edge_features, W_nr, b_nr, W_er, b_er, eps, W1, b1, W2, b2, W_pred, b_pred, edge_index, sub_edge_index):
    raise NotImplementedError("write your pallas kernel here")


# speedup vs baseline: 5.3322x; 5.3322x over previous
import functools

import jax
import jax.numpy as jnp
from jax import lax
from jax.experimental import pallas as pl
from jax.experimental.pallas import tpu as pltpu
from jax.experimental.pallas import tpu_sc as plsc

_N_NODES = 10000
_N_EDGES = 320000
_N_SUB = 65536
_D_NODE = 128
_D_IN = 64

_NC = 2
_NS = 16
_NW = _NC * _NS

_EPW = _N_EDGES // _NW
_CHUNK = 80
_NCHUNK = _EPW // _CHUNK
_N_PAD = 10240
_RPT = _N_PAD // _NS
_ZR = 128

_SPW = _N_SUB // _NW


def _mm_relu_body(x_ref, w_ref, b_ref, o_ref):
    acc = jnp.dot(x_ref[...], w_ref[...], preferred_element_type=jnp.float32)
    o_ref[...] = jnp.maximum(acc + b_ref[...], 0.0)


def _mm_relu(x, w, b, blk):
    rows, k = x.shape
    n = w.shape[1]
    return pl.pallas_call(
        _mm_relu_body,
        grid=(rows // blk,),
        in_specs=[
            pl.BlockSpec((blk, k), lambda i: (i, 0)),
            pl.BlockSpec((k, n), lambda i: (0, 0)),
            pl.BlockSpec((1, n), lambda i: (0, 0)),
        ],
        out_specs=pl.BlockSpec((blk, n), lambda i: (i, 0)),
        out_shape=jax.ShapeDtypeStruct((rows, n), jnp.float32),
    )(x, w, b.reshape(1, n))


def _q_body(x_ref, w_ref, b_ref, wp_ref, bp_ref, o_ref):
    e = jnp.dot(x_ref[...], w_ref[...], preferred_element_type=jnp.float32)
    e = jnp.maximum(e + b_ref[...], 0.0)
    o_ref[...] = jnp.dot(e, wp_ref[...], preferred_element_type=jnp.float32) + bp_ref[...]


def _q_call(x, w, b, wp, bp, blk):
    rows, k = x.shape
    n = w.shape[1]
    return pl.pallas_call(
        _q_body,
        grid=(rows // blk,),
        in_specs=[
            pl.BlockSpec((blk, k), lambda i: (i, 0)),
            pl.BlockSpec((k, n), lambda i: (0, 0)),
            pl.BlockSpec((1, n), lambda i: (0, 0)),
            pl.BlockSpec((n, 1), lambda i: (0, 0)),
            pl.BlockSpec((1, 1), lambda i: (0, 0)),
        ],
        out_specs=pl.BlockSpec((blk, 1), lambda i: (i, 0)),
        out_shape=jax.ShapeDtypeStruct((rows, 1), jnp.float32),
    )(x, w, b.reshape(1, n), wp, bp.reshape(1, 1))


def _head_body(coeff_ref, n_ref, a0_ref, a1_ref, w1_ref, b1_ref, w2_ref,
               b2_ref, wsd_ref, o_ref):
    h = coeff_ref[0, 0] * n_ref[...] + a0_ref[...] + a1_ref[...]
    h = jnp.dot(h, w1_ref[...], preferred_element_type=jnp.float32)
    h = jnp.maximum(h + b1_ref[...], 0.0)
    h = jnp.dot(h, w2_ref[...], preferred_element_type=jnp.float32) + b2_ref[...]
    o_ref[...] = jnp.dot(h, wsd_ref[...], preferred_element_type=jnp.float32)


def _head_call(coeff, n, a0, a1, w1, b1, w2, b2, wsd, blk):
    rows, d = n.shape
    return pl.pallas_call(
        _head_body,
        grid=(rows // blk,),
        in_specs=[
            pl.BlockSpec(memory_space=pltpu.SMEM),
            pl.BlockSpec((blk, d), lambda i: (i, 0)),
            pl.BlockSpec((blk, d), lambda i: (i, 0)),
            pl.BlockSpec((blk, d), lambda i: (i, 0)),
            pl.BlockSpec((d, d), lambda i: (0, 0)),
            pl.BlockSpec((1, d), lambda i: (0, 0)),
            pl.BlockSpec((d, d), lambda i: (0, 0)),
            pl.BlockSpec((1, d), lambda i: (0, 0)),
            pl.BlockSpec((d, 2), lambda i: (0, 0)),
        ],
        out_specs=pl.BlockSpec((blk, 2), lambda i: (i, 0)),
        out_shape=jax.ShapeDtypeStruct((rows, 2), jnp.float32),
    )(coeff, n, a0, a1, w1, b1.reshape(1, d), w2, b2.reshape(1, d), wsd)


_NBUF = 4


def _mp_call(n, e, src2d, dst2d):
    mesh = plsc.VectorSubcoreMesh(core_axis_name="c", subcore_axis_name="s")

    scratch = [
        pltpu.VMEM((_NCHUNK, _CHUNK), jnp.int32),
        pltpu.VMEM((_NCHUNK, _CHUNK), jnp.int32),
    ]
    scratch += [pltpu.VMEM((_CHUNK, _D_IN), jnp.float32) for _ in range(_NBUF)]
    scratch += [pltpu.VMEM((_CHUNK, _D_IN), jnp.float32) for _ in range(_NBUF)]
    scratch += [pltpu.VMEM((_ZR, _D_IN), jnp.float32)]
    scratch += [pltpu.VMEM_SHARED((_N_PAD, _D_IN), jnp.float32)]
    scratch += [pltpu.SemaphoreType.DMA for _ in range(2 * _NBUF)]

    @functools.partial(
        pl.kernel,
        out_type=jax.ShapeDtypeStruct((_NC, _N_PAD, _D_IN), jnp.float32),
        mesh=mesh,
        scratch_types=scratch,
        compiler_params=pltpu.CompilerParams(use_tc_tiling_on_sc=False),
    )
    def body(n_hbm, e_hbm, src_hbm, dst_hbm, out_hbm, *refs):
        srcv, dstv = refs[0], refs[1]
        nb = list(refs[2:2 + _NBUF])
        eb = list(refs[2 + _NBUF:2 + 2 * _NBUF])
        zbuf = refs[2 + 2 * _NBUF]
        acc = refs[3 + 2 * _NBUF]
        sem_in = list(refs[4 + 2 * _NBUF:4 + 3 * _NBUF])
        sem_out = list(refs[4 + 3 * _NBUF:4 + 4 * _NBUF])

        c = lax.axis_index("c")
        s = lax.axis_index("s")

        def zb_body(i, _):
            for j in range(_D_IN // 16):
                zbuf[i, pl.ds(j * 16, 16)] = jnp.zeros((16,), jnp.float32)
            return 0
        lax.fori_loop(0, _ZR, zb_body, 0)
        row0 = s * _RPT
        for kk in range(_RPT // _ZR):
            pltpu.sync_copy(zbuf, acc.at[pl.ds(row0 + kk * _ZR, _ZR)])
        plsc.subcore_barrier()

        wid = c * _NS + s
        base_e = wid * _EPW
        pltpu.sync_copy(src_hbm.at[pl.ds(wid * _NCHUNK, _NCHUNK)], srcv)
        pltpu.sync_copy(dst_hbm.at[pl.ds(wid * _NCHUNK, _NCHUNK)], dstv)

        def issue_in(i, b):
            pltpu.async_copy(e_hbm.at[pl.ds(base_e + i * _CHUNK, _CHUNK)],
                             eb[b], sem_in[b])
            pltpu.async_copy(n_hbm.at[srcv.at[i]], nb[b], sem_in[b])

        def wait_in(i, b):
            pltpu.make_async_copy(e_hbm.at[pl.ds(base_e + i * _CHUNK, _CHUNK)],
                                  eb[b], sem_in[b]).wait()
            pltpu.make_async_copy(n_hbm.at[srcv.at[i]], nb[b], sem_in[b]).wait()

        def compute(b):
            def inner(t, _):
                for dr in range(4):
                    r = t * 4 + dr
                    for g in range(_D_IN // 16):
                        sl = pl.ds(g * 16, 16)
                        eb[b][r, sl] = jnp.maximum(nb[b][r, sl] + eb[b][r, sl], 0.0)
                return 0
            lax.fori_loop(0, _CHUNK // 4, inner, 0)

        def issue_out(i, b):
            pltpu.async_copy(eb[b], acc.at[dstv.at[i]], sem_out[b], add=True)

        def wait_out(i, b):
            pltpu.make_async_copy(eb[b], acc.at[dstv.at[i]], sem_out[b]).wait()

        for b in range(_NBUF):
            issue_in(b, b)

        def quad(j, _):
            for b in range(_NBUF):
                i = j * _NBUF + b
                wait_in(i, b)
                compute(b)
                issue_out(i, b)

                @pl.when(i + _NBUF < _NCHUNK)
                def _():
                    wait_out(i, b)
                    issue_in(i + _NBUF, b)
            return 0
        lax.fori_loop(0, _NCHUNK // _NBUF, quad, 0)

        i_last = _NCHUNK - 1
        wait_in(i_last, 0)
        compute(0)
        issue_out(i_last, 0)
        for b in range(_NBUF):
            wait_out(0, b)

        plsc.subcore_barrier()
        pltpu.sync_copy(acc.at[pl.ds(row0, _RPT)], out_hbm.at[c, pl.ds(row0, _RPT)])

    return body(n, e, src2d, dst2d)


def _pred_call(ps, pd, s_idx, d_idx, q):
    mesh = plsc.VectorSubcoreMesh(core_axis_name="c", subcore_axis_name="s")

    @functools.partial(
        pl.kernel,
        out_type=jax.ShapeDtypeStruct((_N_SUB,), jnp.float32),
        mesh=mesh,
        scratch_types=[
            pltpu.VMEM((_N_NODES,), jnp.float32),
            pltpu.VMEM((_N_NODES,), jnp.float32),
            pltpu.VMEM((_SPW,), jnp.int32),
            pltpu.VMEM((_SPW,), jnp.int32),
            pltpu.VMEM((_SPW,), jnp.float32),
            pltpu.VMEM((_SPW,), jnp.float32),
        ],
        compiler_params=pltpu.CompilerParams(use_tc_tiling_on_sc=False,
                                             needs_layout_passes=False),
    )
    def body(ps_hbm, pd_hbm, s_hbm, d_hbm, q_hbm, out_hbm,
             psv, pdv, sv, dv, qv, ov):
        c = lax.axis_index("c")
        s = lax.axis_index("s")
        base = (c * _NS + s) * _SPW
        pltpu.sync_copy(ps_hbm, psv)
        pltpu.sync_copy(pd_hbm, pdv)
        pltpu.sync_copy(s_hbm.at[pl.ds(base, _SPW)], sv)
        pltpu.sync_copy(d_hbm.at[pl.ds(base, _SPW)], dv)
        pltpu.sync_copy(q_hbm.at[pl.ds(base, _SPW)], qv)

        def it(k, _):
            sl = pl.ds(k * 16, 16)
            vs = plsc.load_gather(psv, [sv[sl]])
            vd = plsc.load_gather(pdv, [dv[sl]])
            ov[sl] = vs + vd + qv[sl]
            return 0
        lax.fori_loop(0, _SPW // 16, it, 0)
        pltpu.sync_copy(ov, out_hbm.at[pl.ds(base, _SPW)])

    return body(ps, pd, s_idx, d_idx, q)


def kernel(node_features, edge_features, sub_edge_features,
           W_nr, b_nr, W_er, b_er, eps, W1, b1, W2, b2, W_pred, b_pred,
           edge_index, sub_edge_index):
    src = edge_index[0].astype(jnp.int32).reshape(_NW * _NCHUNK, _CHUNK)
    dst = edge_index[1].astype(jnp.int32).reshape(_NW * _NCHUNK, _CHUNK)
    s_sub = sub_edge_index[0].astype(jnp.int32)
    d_sub = sub_edge_index[1].astype(jnp.int32)

    n = _mm_relu(node_features, W_nr, b_nr, blk=2000)
    e = _mm_relu(edge_features, W_er, b_er, blk=8000)
    q = _q_call(sub_edge_features, W_er, b_er, W_pred[2 * _D_IN:], b_pred,
                blk=8192)

    agg = _mp_call(n, e, src, dst)[:, :_N_NODES, :]

    coeff = jnp.reshape(1.0 + eps, (1, 1)).astype(jnp.float32)
    wsd = jnp.concatenate([W_pred[:_D_IN], W_pred[_D_IN:2 * _D_IN]], axis=1)
    ps_pd = _head_call(coeff, n, agg[0], agg[1], W1, b1, W2, b2, wsd, blk=2000)

    out = _pred_call(ps_pd[:, 0], ps_pd[:, 1],
                     s_sub, d_sub, q.reshape(_N_SUB))
    return out.reshape(_N_SUB, 1)
